# baseline (speedup 1.0000x reference)
import functools

import jax
import jax.numpy as jnp
from jax import lax
from jax.experimental import pallas as pl
from jax.experimental.pallas import tpu as pltpu
from jax.experimental.pallas import tpu_sc as plsc

N_USERS = 5000
N_ITEMS = 5000
NT = N_USERS + N_ITEMS
D = 128
E = 320000
N_LAYERS = 3
EPS = 1e-7

NC = 2
NS = 16
NW = NC * NS

C = 128
EPW = 10000
CH = 80
NPAD = 10240
RPT = 640
NSPLIT = 4
SR = C // NSPLIT

_mesh = plsc.VectorSubcoreMesh(core_axis_name="c", subcore_axis_name="s")


@functools.partial(
    pl.kernel,
    out_type=jax.ShapeDtypeStruct((NC, NPAD, 16), jnp.float32),
    mesh=_mesh,
    scratch_types=[
        pltpu.VMEM((2, 2, C), jnp.int32),
        pltpu.VMEM((C, 16), jnp.float32),
        pltpu.VMEM((C, 16), jnp.float32),
        pltpu.VMEM_SHARED((NPAD, 16), jnp.float32),
        pltpu.SemaphoreType.DMA,
        pltpu.SemaphoreType.DMA,
    ],
)
def _k0_degrees(idx_hbm, deg_hbm, ibuf, ones_v, zrow_v, deg_sp, si0, si1):
    cid = lax.axis_index("c")
    sid = lax.axis_index("s")
    wid = cid * NS + sid
    sis = (si0, si1)

    one_row = jnp.full((16,), 1.0, jnp.float32)
    zrow = jnp.zeros((16,), jnp.float32)

    def init_rows(i, _):
        ones_v[i, :] = one_row
        zrow_v[i, :] = zrow
        return 0
    lax.fori_loop(0, C, init_rows, 0)

    base = sid * RPT
    for k in range(RPT // C):
        pltpu.sync_copy(zrow_v, deg_sp.at[pl.ds(base + k * C, C)])
    plsc.subcore_barrier()

    pltpu.sync_copy(idx_hbm.at[wid, 0], ibuf.at[0])

    def body(jj, _):
        for b in range(2):
            j = jj * 2 + b

            @pl.when(j > 0)
            def _wait():
                pltpu.make_async_copy(
                    idx_hbm.at[wid, j], ibuf.at[b], sis[b]).wait()

            @pl.when(j + 1 < CH)
            def _issue():
                pltpu.async_copy(
                    idx_hbm.at[wid, j + 1], ibuf.at[1 - b], sis[1 - b])

            pltpu.sync_copy(ones_v, deg_sp.at[ibuf.at[b, 1]], add=True)
        return 0

    lax.fori_loop(0, CH // 2, body, 0)
    plsc.subcore_barrier()

    out_base = pl.multiple_of(sid * RPT, 8)
    pltpu.sync_copy(deg_sp.at[pl.ds(out_base, RPT)],
                    deg_hbm.at[cid, pl.ds(out_base, RPT)])


@functools.partial(
    pl.kernel,
    out_type=jax.ShapeDtypeStruct((NC, NT, D), jnp.float32),
    mesh=_mesh,
    scratch_types=[
        pltpu.VMEM((2, 2, C), jnp.int32),
        pltpu.VMEM((2, C, D), jnp.float32),
        pltpu.VMEM_SHARED((NT + 8, D), jnp.float32),
        pltpu.SemaphoreType.DMA,
        pltpu.SemaphoreType.DMA,
        pltpu.SemaphoreType.DMA,
        pltpu.SemaphoreType.DMA,
    ],
)
def _k1_propagate(z_hbm, idx_hbm, out_hbm,
                  ibuf, rows_v, acc_sp, si0, si1, sg0, sg1):
    cid = lax.axis_index("c")
    sid = lax.axis_index("s")
    wid = cid * NS + sid
    sis = (si0, si1)
    sgs = (sg0, sg1)

    zv = jnp.zeros((16,), jnp.float32)

    def zbody(i, _):
        r = i // (D // 16)
        col = (i % (D // 16)) * 16
        rows_v[0, r, pl.ds(col, 16)] = zv
        return 0
    lax.fori_loop(0, C * (D // 16), zbody, 0)

    base = pl.multiple_of(jnp.minimum(sid * RPT, NT - RPT), 8)
    for k in range(RPT // C):
        pltpu.sync_copy(rows_v.at[0], acc_sp.at[pl.ds(base + k * C, C)])
    plsc.subcore_barrier()

    def _gather(b):
        for g in range(NSPLIT):
            pltpu.async_copy(
                z_hbm.at[ibuf.at[b, 0, pl.ds(g * SR, SR)]],
                rows_v.at[b, pl.ds(g * SR, SR)], sgs[b])

    pltpu.sync_copy(idx_hbm.at[wid, 0], ibuf.at[0])
    _gather(0)
    pltpu.async_copy(idx_hbm.at[wid, 1], ibuf.at[1], si1)

    def body(jj, _):
        for b in range(2):
            j = jj * 2 + b

            @pl.when(j + 1 < CH)
            def _gather_next():
                pltpu.make_async_copy(
                    idx_hbm.at[wid, j + 1], ibuf.at[1 - b], sis[1 - b]).wait()
                _gather(1 - b)

            pltpu.make_async_copy(
                z_hbm.at[ibuf.at[b, 0]], rows_v.at[b], sgs[b]).wait()
            pltpu.sync_copy(rows_v.at[b], acc_sp.at[ibuf.at[b, 1]], add=True)

            @pl.when(j + 2 < CH)
            def _prefetch_idx():
                pltpu.async_copy(idx_hbm.at[wid, j + 2], ibuf.at[b], sis[b])
        return 0

    lax.fori_loop(0, CH // 2, body, 0)
    plsc.subcore_barrier()

    pltpu.sync_copy(acc_sp.at[pl.ds(base, RPT)],
                    out_hbm.at[cid, pl.ds(base, RPT)])


_BR = 1000
_row_spec = pl.BlockSpec((_BR, D), lambda i: (i, 0))
_deg_spec = pl.BlockSpec((NC, _BR, 16), lambda i: (0, i, 0))


def _deg_col(deg_ref):
    return deg_ref[0, :, 0:1] + deg_ref[1, :, 0:1] + EPS


def _tc_scale_body(x_ref, deg_ref, o_ref):
    o_ref[...] = x_ref[...] * lax.rsqrt(_deg_col(deg_ref))


_tc_scale = pl.pallas_call(
    _tc_scale_body,
    grid=(NT // _BR,),
    in_specs=[_row_spec, _deg_spec],
    out_specs=_row_spec,
    out_shape=jax.ShapeDtypeStruct((NT, D), jnp.float32),
)


def _tc_combine_body(p0_ref, p1_ref, deg_ref, o_ref):
    o_ref[...] = (p0_ref[...] + p1_ref[...]) / _deg_col(deg_ref)


_tc_combine = pl.pallas_call(
    _tc_combine_body,
    grid=(NT // _BR,),
    in_specs=[_row_spec, _row_spec, _deg_spec],
    out_specs=_row_spec,
    out_shape=jax.ShapeDtypeStruct((NT, D), jnp.float32),
)


def _tc_final_body(z0_ref, z1_ref, z2_ref, p0_ref, p1_ref, deg_ref, o_ref):
    d = _deg_col(deg_ref)
    z3 = (p0_ref[...] + p1_ref[...]) / d
    o_ref[...] = ((z0_ref[...] + z1_ref[...] + z2_ref[...] + z3)
                  * (0.25 * lax.sqrt(d)))


_tc_final = pl.pallas_call(
    _tc_final_body,
    grid=(NT // _BR,),
    in_specs=[_row_spec, _row_spec, _row_spec, _row_spec, _row_spec,
              _deg_spec],
    out_specs=_row_spec,
    out_shape=jax.ShapeDtypeStruct((NT, D), jnp.float32),
)


def kernel(user_emb, item_emb, edge_index):
    src = edge_index[0]
    dst = edge_index[1]

    pad = CH * C - EPW
    src2 = jnp.pad(src.reshape(NW, EPW), ((0, 0), (0, pad)),
                   constant_values=0)
    dst2 = jnp.pad(dst.reshape(NW, EPW), ((0, 0), (0, pad)),
                   constant_values=NT)
    idx = jnp.stack([src2.reshape(NW, CH, C), dst2.reshape(NW, CH, C)],
                    axis=2)

    deg16 = _k0_degrees(idx)[:, :NT]

    all_emb = jnp.concatenate([user_emb, item_emb], axis=0)
    z0 = _tc_scale(all_emb, deg16)

    z = z0
    zs = [z0]
    for _ in range(N_LAYERS - 1):
        p = _k1_propagate(z, idx)
        z = _tc_combine(p[0], p[1], deg16)
        zs.append(z)

    p = _k1_propagate(z, idx)
    out = _tc_final(zs[0], zs[1], zs[2], p[0], p[1], deg16)

    return (out[:N_USERS], out[N_USERS:])


# speedup vs baseline: 1.0715x; 1.0715x over previous
import functools

import jax
import jax.numpy as jnp
from jax import lax
from jax.experimental import pallas as pl
from jax.experimental.pallas import tpu as pltpu
from jax.experimental.pallas import tpu_sc as plsc

N_USERS = 5000
N_ITEMS = 5000
NT = N_USERS + N_ITEMS
NH = NT // 2
D = 128
E = 320000
EH = E // 2
N_LAYERS = 3
EPS = 1e-7

NC = 2
NS = 16
NW = NC * NS

C = 256
EPW = EH // NS
CH = 40
RPT = 320
DROW = NH

_mesh = plsc.VectorSubcoreMesh(core_axis_name="c", subcore_axis_name="s")


@functools.partial(
    pl.kernel,
    out_type=jax.ShapeDtypeStruct((NC, NS * RPT, 16), jnp.float32),
    mesh=_mesh,
    scratch_types=[
        pltpu.VMEM((2, 2, 2, 128), jnp.int32),
        pltpu.VMEM((128, 16), jnp.float32),
        pltpu.VMEM((128, 16), jnp.float32),
        pltpu.VMEM_SHARED((NS * RPT, 16), jnp.float32),
        pltpu.SemaphoreType.DMA,
        pltpu.SemaphoreType.DMA,
    ],
)
def _k0_degrees(idx_hbm, deg_hbm, ibuf, ones_v, zrow_v, deg_sp, si0, si1):
    cid = lax.axis_index("c")
    sid = lax.axis_index("s")
    wid = cid * NS + sid
    sis = (si0, si1)

    one_row = jnp.full((16,), 1.0, jnp.float32)
    zrow = jnp.zeros((16,), jnp.float32)

    def init_rows(i, _):
        ones_v[i, :] = one_row
        zrow_v[i, :] = zrow
        return 0
    lax.fori_loop(0, 128, init_rows, 0)

    base = sid * RPT
    for k in range(RPT // 128):
        pltpu.sync_copy(zrow_v, deg_sp.at[pl.ds(base + k * 128, 128)])
    pltpu.sync_copy(zrow_v.at[pl.ds(0, 64)],
                    deg_sp.at[pl.ds(base + 256, 64)])
    plsc.subcore_barrier()

    pltpu.sync_copy(idx_hbm.at[wid, 0], ibuf.at[0])

    def body(jj, _):
        for b in range(2):
            j = jj * 2 + b

            @pl.when(j > 0)
            def _wait():
                pltpu.make_async_copy(
                    idx_hbm.at[wid, j], ibuf.at[b], sis[b]).wait()

            @pl.when(j + 1 < CH)
            def _issue():
                pltpu.async_copy(
                    idx_hbm.at[wid, j + 1], ibuf.at[1 - b], sis[1 - b])

            for k in range(2):
                pltpu.sync_copy(ones_v, deg_sp.at[ibuf.at[b, 1, k]],
                                add=True)
        return 0

    lax.fori_loop(0, CH // 2, body, 0)
    plsc.subcore_barrier()

    out_base = pl.multiple_of(sid * RPT, 8)
    pltpu.sync_copy(deg_sp.at[pl.ds(out_base, RPT)],
                    deg_hbm.at[cid, pl.ds(out_base, RPT)])


@functools.partial(
    pl.kernel,
    out_type=jax.ShapeDtypeStruct((NT, D), jnp.float32),
    mesh=_mesh,
    scratch_types=[
        pltpu.VMEM((CH, 2, 2, 128), jnp.int32),
        pltpu.VMEM((2, C, D), jnp.float32),
        pltpu.VMEM_SHARED((NH + 8, D), jnp.float32),
        pltpu.SemaphoreType.DMA,
        pltpu.SemaphoreType.DMA,
    ],
)
def _k1_propagate(z_hbm, idx_hbm, out_hbm, ipre, rows_v, acc_sp, sg0, sg1):
    cid = lax.axis_index("c")
    sid = lax.axis_index("s")
    wid = cid * NS + sid
    sgs = (sg0, sg1)

    zv = jnp.zeros((16,), jnp.float32)

    def zbody(i, _):
        r = i // (D // 16)
        col = (i % (D // 16)) * 16
        rows_v[0, r, pl.ds(col, 16)] = zv
        return 0
    lax.fori_loop(0, C * (D // 16), zbody, 0)

    base = pl.multiple_of(jnp.minimum(sid * RPT, NH - RPT), 8)
    pltpu.sync_copy(rows_v.at[0], acc_sp.at[pl.ds(base, C)])
    pltpu.sync_copy(rows_v.at[0, pl.ds(0, RPT - C)],
                    acc_sp.at[pl.ds(base + C, RPT - C)])
    plsc.subcore_barrier()

    pltpu.sync_copy(idx_hbm.at[wid], ipre)

    def _gather(j, b):
        for k in range(2):
            pltpu.async_copy(
                z_hbm.at[ipre.at[j, 0, k]],
                rows_v.at[b, pl.ds(k * 128, 128)], sgs[b])

    def _wait_gather(j, b):
        for k in range(2):
            pltpu.make_async_copy(
                z_hbm.at[ipre.at[j, 0, k]],
                rows_v.at[b, pl.ds(k * 128, 128)], sgs[b]).wait()

    _gather(0, 0)
    _gather(1, 1)

    def body(jj, _):
        for b in range(2):
            j = jj * 2 + b
            _wait_gather(j, b)
            for k in range(2):
                pltpu.sync_copy(rows_v.at[b, pl.ds(k * 128, 128)],
                                acc_sp.at[ipre.at[j, 1, k]], add=True)

            @pl.when(j + 2 < CH)
            def _next():
                _gather(j + 2, b)
        return 0

    lax.fori_loop(0, CH // 2, body, 0)
    plsc.subcore_barrier()

    pltpu.sync_copy(acc_sp.at[pl.ds(base, RPT)],
                    out_hbm.at[pl.ds(cid * NH + base, RPT)])


_BR = 1000
_row_spec = pl.BlockSpec((_BR, D), lambda i: (i, 0))
_deg_spec = pl.BlockSpec((_BR, 16), lambda i: (i, 0))


def _deg_col(deg_ref):
    return deg_ref[:, 0:1] + EPS


def _tc_scale_body(x_ref, deg_ref, o_ref):
    o_ref[...] = x_ref[...] * lax.rsqrt(_deg_col(deg_ref))


_tc_scale = pl.pallas_call(
    _tc_scale_body,
    grid=(NT // _BR,),
    in_specs=[_row_spec, _deg_spec],
    out_specs=_row_spec,
    out_shape=jax.ShapeDtypeStruct((NT, D), jnp.float32),
)


def _tc_combine_body(u_ref, deg_ref, o_ref):
    o_ref[...] = u_ref[...] / _deg_col(deg_ref)


_tc_combine = pl.pallas_call(
    _tc_combine_body,
    grid=(NT // _BR,),
    in_specs=[_row_spec, _deg_spec],
    out_specs=_row_spec,
    out_shape=jax.ShapeDtypeStruct((NT, D), jnp.float32),
)


def _tc_final_body(z0_ref, z1_ref, z2_ref, u_ref, deg_ref, o_ref):
    d = _deg_col(deg_ref)
    z3 = u_ref[...] / d
    o_ref[...] = ((z0_ref[...] + z1_ref[...] + z2_ref[...] + z3)
                  * (0.25 * lax.sqrt(d)))


_tc_final = pl.pallas_call(
    _tc_final_body,
    grid=(NT // _BR,),
    in_specs=[_row_spec, _row_spec, _row_spec, _row_spec, _deg_spec],
    out_specs=_row_spec,
    out_shape=jax.ShapeDtypeStruct((NT, D), jnp.float32),
)


def kernel(user_emb, item_emb, edge_index):
    src = edge_index[0]
    dst = edge_index[1]

    srcH = jnp.concatenate([src[EH:], src[:EH]]).reshape(NW, EPW)
    dstH = jnp.concatenate([dst[EH:], dst[:EH] - NH]).reshape(NW, EPW)
    pad = CH * C - EPW
    src2 = jnp.pad(srcH, ((0, 0), (0, pad)), constant_values=0)
    dst2 = jnp.pad(dstH, ((0, 0), (0, pad)), constant_values=DROW)
    idx = jnp.stack([src2.reshape(NW, CH, 2, 128),
                     dst2.reshape(NW, CH, 2, 128)], axis=2)

    deg2 = _k0_degrees(idx)
    deg16 = jnp.concatenate([deg2[0, :NH], deg2[1, :NH]], axis=0)

    all_emb = jnp.concatenate([user_emb, item_emb], axis=0)
    z0 = _tc_scale(all_emb, deg16)

    z = z0
    zs = [z0]
    for _ in range(N_LAYERS - 1):
        u = _k1_propagate(z, idx)
        z = _tc_combine(u, deg16)
        zs.append(z)

    u = _k1_propagate(z, idx)
    out = _tc_final(zs[0], zs[1], zs[2], u, deg16)

    return (out[:N_USERS], out[N_USERS:])


# speedup vs baseline: 2.1092x; 1.9684x over previous
import functools

import jax
import jax.numpy as jnp
from jax import lax
from jax.experimental import pallas as pl
from jax.experimental.pallas import tpu as pltpu
from jax.experimental.pallas import tpu_sc as plsc

N_USERS = 5000
N_ITEMS = 5000
NT = N_USERS + N_ITEMS
NH = NT // 2
D = 128
E = 320000
EH = E // 2
N_LAYERS = 3
EPS = 1e-7

NC = 2
NS = 16
NW = NC * NS

C = 128
EPW = EH // NS
CH = 80
RPT = 320
DROW = NH

_mesh = plsc.VectorSubcoreMesh(core_axis_name="c", subcore_axis_name="s")


@functools.partial(
    pl.kernel,
    out_type=jax.ShapeDtypeStruct((NC, NS * RPT, 16), jnp.float32),
    mesh=_mesh,
    scratch_types=[
        pltpu.VMEM((2, 2, 128), jnp.int32),
        pltpu.VMEM((128, 16), jnp.float32),
        pltpu.VMEM((128, 16), jnp.float32),
        pltpu.VMEM_SHARED((NS * RPT, 16), jnp.float32),
        pltpu.SemaphoreType.DMA,
        pltpu.SemaphoreType.DMA,
    ],
)
def _k0_degrees(idx_hbm, deg_hbm, ibuf, ones_v, zrow_v, deg_sp, si0, si1):
    cid = lax.axis_index("c")
    sid = lax.axis_index("s")
    wid = cid * NS + sid
    sis = (si0, si1)

    one_row = jnp.full((16,), 1.0, jnp.float32)
    zrow = jnp.zeros((16,), jnp.float32)

    def init_rows(i, _):
        ones_v[i, :] = one_row
        zrow_v[i, :] = zrow
        return 0
    lax.fori_loop(0, 128, init_rows, 0)

    base = sid * RPT
    for k in range(RPT // 128):
        pltpu.sync_copy(zrow_v, deg_sp.at[pl.ds(base + k * 128, 128)])
    pltpu.sync_copy(zrow_v.at[pl.ds(0, 64)],
                    deg_sp.at[pl.ds(base + 256, 64)])
    plsc.subcore_barrier()

    pltpu.sync_copy(idx_hbm.at[wid, 0], ibuf.at[0])

    def body(jj, _):
        for b in range(2):
            j = jj * 2 + b

            @pl.when(j > 0)
            def _wait():
                pltpu.make_async_copy(
                    idx_hbm.at[wid, j], ibuf.at[b], sis[b]).wait()

            @pl.when(j + 1 < CH)
            def _issue():
                pltpu.async_copy(
                    idx_hbm.at[wid, j + 1], ibuf.at[1 - b], sis[1 - b])

            pltpu.sync_copy(ones_v, deg_sp.at[ibuf.at[b, 1]], add=True)
        return 0

    lax.fori_loop(0, CH // 2, body, 0)
    plsc.subcore_barrier()

    out_base = pl.multiple_of(sid * RPT, 8)
    pltpu.sync_copy(deg_sp.at[pl.ds(out_base, RPT)],
                    deg_hbm.at[cid, pl.ds(out_base, RPT)])


@functools.partial(
    pl.kernel,
    out_type=jax.ShapeDtypeStruct((NT, D), jnp.float32),
    mesh=_mesh,
    scratch_types=[
        pltpu.VMEM((2, 2, 128), jnp.int32),
        pltpu.VMEM((2, C, D), jnp.float32),
        pltpu.VMEM_SHARED((NH, D), jnp.float32),
        pltpu.VMEM_SHARED((NH + 8, D), jnp.float32),
        pltpu.SemaphoreType.DMA,
        pltpu.SemaphoreType.DMA,
        pltpu.SemaphoreType.DMA,
        pltpu.SemaphoreType.DMA,
    ],
)
def _k1_propagate(z_hbm, idx_hbm, out_hbm, ibuf, rows_v, zsp, acc_sp,
                  si0, si1, sg0, sg1):
    cid = lax.axis_index("c")
    sid = lax.axis_index("s")
    wid = cid * NS + sid
    sis = (si0, si1)
    sgs = (sg0, sg1)

    base = pl.multiple_of(jnp.minimum(sid * RPT, NH - RPT), 8)

    src_off = pl.multiple_of((1 - cid) * NH + base, 8)
    pltpu.sync_copy(z_hbm.at[pl.ds(src_off, RPT)], zsp.at[pl.ds(base, RPT)])

    zv = jnp.zeros((16,), jnp.float32)

    def zbody(i, _):
        r = i // (D // 16)
        col = (i % (D // 16)) * 16
        rows_v[0, r, pl.ds(col, 16)] = zv
        return 0
    lax.fori_loop(0, C * (D // 16), zbody, 0)

    for k in range(RPT // C):
        pltpu.sync_copy(rows_v.at[0], acc_sp.at[pl.ds(base + k * C, C)])
    pltpu.sync_copy(rows_v.at[0, pl.ds(0, RPT % C)],
                    acc_sp.at[pl.ds(base + (RPT // C) * C, RPT % C)])
    plsc.subcore_barrier()

    pltpu.sync_copy(idx_hbm.at[wid, 0], ibuf.at[0])
    pltpu.async_copy(zsp.at[ibuf.at[0, 0]], rows_v.at[0], sg0)
    pltpu.async_copy(idx_hbm.at[wid, 1], ibuf.at[1], si1)

    def body(jj, _):
        for b in range(2):
            j = jj * 2 + b

            @pl.when(j + 1 < CH)
            def _gather_next():
                pltpu.make_async_copy(
                    idx_hbm.at[wid, j + 1], ibuf.at[1 - b], sis[1 - b]).wait()
                pltpu.async_copy(
                    zsp.at[ibuf.at[1 - b, 0]], rows_v.at[1 - b], sgs[1 - b])

            pltpu.make_async_copy(
                zsp.at[ibuf.at[b, 0]], rows_v.at[b], sgs[b]).wait()
            pltpu.sync_copy(rows_v.at[b], acc_sp.at[ibuf.at[b, 1]], add=True)

            @pl.when(j + 2 < CH)
            def _prefetch_idx():
                pltpu.async_copy(idx_hbm.at[wid, j + 2], ibuf.at[b], sis[b])
        return 0

    lax.fori_loop(0, CH // 2, body, 0)
    plsc.subcore_barrier()

    pltpu.sync_copy(acc_sp.at[pl.ds(base, RPT)],
                    out_hbm.at[pl.ds(cid * NH + base, RPT)])


_BR = 1000
_row_spec = pl.BlockSpec((_BR, D), lambda i: (i, 0))
_deg_spec = pl.BlockSpec((_BR, 16), lambda i: (i, 0))


def _deg_col(deg_ref):
    return deg_ref[:, 0:1] + EPS


def _tc_scale_body(x_ref, deg_ref, o_ref):
    o_ref[...] = x_ref[...] * lax.rsqrt(_deg_col(deg_ref))


_tc_scale = pl.pallas_call(
    _tc_scale_body,
    grid=(NT // _BR,),
    in_specs=[_row_spec, _deg_spec],
    out_specs=_row_spec,
    out_shape=jax.ShapeDtypeStruct((NT, D), jnp.float32),
)


def _tc_combine_body(u_ref, deg_ref, o_ref):
    o_ref[...] = u_ref[...] / _deg_col(deg_ref)


_tc_combine = pl.pallas_call(
    _tc_combine_body,
    grid=(NT // _BR,),
    in_specs=[_row_spec, _deg_spec],
    out_specs=_row_spec,
    out_shape=jax.ShapeDtypeStruct((NT, D), jnp.float32),
)


def _tc_final_body(z0_ref, z1_ref, z2_ref, u_ref, deg_ref, o_ref):
    d = _deg_col(deg_ref)
    z3 = u_ref[...] / d
    o_ref[...] = ((z0_ref[...] + z1_ref[...] + z2_ref[...] + z3)
                  * (0.25 * lax.sqrt(d)))


_tc_final = pl.pallas_call(
    _tc_final_body,
    grid=(NT // _BR,),
    in_specs=[_row_spec, _row_spec, _row_spec, _row_spec, _deg_spec],
    out_specs=_row_spec,
    out_shape=jax.ShapeDtypeStruct((NT, D), jnp.float32),
)


def kernel(user_emb, item_emb, edge_index):
    src = edge_index[0]
    dst = edge_index[1]

    srcH = jnp.concatenate([src[EH:] - NH, src[:EH]]).reshape(NW, EPW)
    dstH = jnp.concatenate([dst[EH:], dst[:EH] - NH]).reshape(NW, EPW)
    pad = CH * C - EPW
    src2 = jnp.pad(srcH, ((0, 0), (0, pad)), constant_values=0)
    dst2 = jnp.pad(dstH, ((0, 0), (0, pad)), constant_values=DROW)
    idx = jnp.stack([src2.reshape(NW, CH, 128),
                     dst2.reshape(NW, CH, 128)], axis=2)

    deg2 = _k0_degrees(idx)
    deg16 = jnp.concatenate([deg2[0, :NH], deg2[1, :NH]], axis=0)

    all_emb = jnp.concatenate([user_emb, item_emb], axis=0)
    z0 = _tc_scale(all_emb, deg16)

    z = z0
    zs = [z0]
    for _ in range(N_LAYERS - 1):
        u = _k1_propagate(z, idx)
        z = _tc_combine(u, deg16)
        zs.append(z)

    u = _k1_propagate(z, idx)
    out = _tc_final(zs[0], zs[1], zs[2], u, deg16)

    return (out[:N_USERS], out[N_USERS:])


# speedup vs baseline: 2.1250x; 1.0075x over previous
import functools

import jax
import jax.numpy as jnp
from jax import lax
from jax.experimental import pallas as pl
from jax.experimental.pallas import tpu as pltpu
from jax.experimental.pallas import tpu_sc as plsc

N_USERS = 5000
N_ITEMS = 5000
NT = N_USERS + N_ITEMS
NH = NT // 2
NHP = 5120
NT2 = 2 * NHP
D = 128
E = 320000
EH = E // 2
N_LAYERS = 3
EPS = 1e-7

NC = 2
NS = 16
NW = NC * NS

C = 128
EPW = EH // NS
CH = 80
RPT = NHP // NS
DROW = NH

_mesh = plsc.VectorSubcoreMesh(core_axis_name="c", subcore_axis_name="s")


@functools.partial(
    pl.kernel,
    out_type=jax.ShapeDtypeStruct((NC, NS * RPT, 16), jnp.float32),
    mesh=_mesh,
    scratch_types=[
        pltpu.VMEM((2, 2, 128), jnp.int32),
        pltpu.VMEM((128, 16), jnp.float32),
        pltpu.VMEM((128, 16), jnp.float32),
        pltpu.VMEM_SHARED((NS * RPT, 16), jnp.float32),
        pltpu.SemaphoreType.DMA,
        pltpu.SemaphoreType.DMA,
    ],
)
def _k0_degrees(idx_hbm, deg_hbm, ibuf, ones_v, zrow_v, deg_sp, si0, si1):
    cid = lax.axis_index("c")
    sid = lax.axis_index("s")
    wid = cid * NS + sid
    sis = (si0, si1)

    one_row = jnp.full((16,), 1.0, jnp.float32)
    zrow = jnp.zeros((16,), jnp.float32)

    def init_rows(i, _):
        ones_v[i, :] = one_row
        zrow_v[i, :] = zrow
        return 0
    lax.fori_loop(0, 128, init_rows, 0)

    base = sid * RPT
    for k in range(RPT // 128):
        pltpu.sync_copy(zrow_v, deg_sp.at[pl.ds(base + k * 128, 128)])
    pltpu.sync_copy(zrow_v.at[pl.ds(0, 64)],
                    deg_sp.at[pl.ds(base + 256, 64)])
    plsc.subcore_barrier()

    pltpu.sync_copy(idx_hbm.at[wid, 0], ibuf.at[0])

    def body(jj, _):
        for b in range(2):
            j = jj * 2 + b

            @pl.when(j > 0)
            def _wait():
                pltpu.make_async_copy(
                    idx_hbm.at[wid, j], ibuf.at[b], sis[b]).wait()

            @pl.when(j + 1 < CH)
            def _issue():
                pltpu.async_copy(
                    idx_hbm.at[wid, j + 1], ibuf.at[1 - b], sis[1 - b])

            pltpu.sync_copy(ones_v, deg_sp.at[ibuf.at[b, 1]], add=True)
        return 0

    lax.fori_loop(0, CH // 2, body, 0)
    plsc.subcore_barrier()

    out_base = pl.multiple_of(sid * RPT, 8)
    pltpu.sync_copy(deg_sp.at[pl.ds(out_base, RPT)],
                    deg_hbm.at[cid, pl.ds(out_base, RPT)])


@functools.partial(
    pl.kernel,
    out_type=jax.ShapeDtypeStruct((NT2, D), jnp.float32),
    mesh=_mesh,
    scratch_types=[
        pltpu.VMEM((2, 2, 128), jnp.int32),
        pltpu.VMEM((2, C, D), jnp.float32),
        pltpu.VMEM((RPT // 8, 128), jnp.float32),
        pltpu.VMEM_SHARED((NHP, D), jnp.float32),
        pltpu.VMEM_SHARED((NHP, D), jnp.float32),
        pltpu.SemaphoreType.DMA,
        pltpu.SemaphoreType.DMA,
        pltpu.SemaphoreType.DMA,
        pltpu.SemaphoreType.DMA,
    ],
)
def _k1_propagate(z_hbm, idx_hbm, deg_hbm, out_hbm, ibuf, rows_v, degv,
                  zsp, acc_sp, si0, si1, sg0, sg1):
    cid = lax.axis_index("c")
    sid = lax.axis_index("s")
    wid = cid * NS + sid
    sis = (si0, si1)
    sgs = (sg0, sg1)

    base = pl.multiple_of(sid * RPT, 8)

    src_off = pl.multiple_of((1 - cid) * NHP + base, 8)
    pltpu.sync_copy(z_hbm.at[pl.ds(src_off, RPT)], zsp.at[pl.ds(base, RPT)])
    dst_off = pl.multiple_of(cid * NHP + base, 8)
    pltpu.sync_copy(
        deg_hbm.at[pl.ds(pl.multiple_of(dst_off // 8, 8), RPT // 8)], degv)

    zv = jnp.zeros((16,), jnp.float32)

    def zbody(i, _):
        r = i // (D // 16)
        col = (i % (D // 16)) * 16
        rows_v[0, r, pl.ds(col, 16)] = zv
        return 0
    lax.fori_loop(0, C * (D // 16), zbody, 0)

    for k in range(RPT // C):
        pltpu.sync_copy(rows_v.at[0], acc_sp.at[pl.ds(base + k * C, C)])
    pltpu.sync_copy(rows_v.at[0, pl.ds(0, RPT % C)],
                    acc_sp.at[pl.ds(base + (RPT // C) * C, RPT % C)])
    plsc.subcore_barrier()

    pltpu.sync_copy(idx_hbm.at[wid, 0], ibuf.at[0])
    pltpu.async_copy(zsp.at[ibuf.at[0, 0]], rows_v.at[0], sg0)
    pltpu.async_copy(idx_hbm.at[wid, 1], ibuf.at[1], si1)

    def body(jj, _):
        for b in range(2):
            j = jj * 2 + b

            @pl.when(j + 1 < CH)
            def _gather_next():
                pltpu.make_async_copy(
                    idx_hbm.at[wid, j + 1], ibuf.at[1 - b], sis[1 - b]).wait()
                pltpu.async_copy(
                    zsp.at[ibuf.at[1 - b, 0]], rows_v.at[1 - b], sgs[1 - b])

            pltpu.make_async_copy(
                zsp.at[ibuf.at[b, 0]], rows_v.at[b], sgs[b]).wait()
            pltpu.sync_copy(rows_v.at[b], acc_sp.at[ibuf.at[b, 1]], add=True)

            @pl.when(j + 2 < CH)
            def _prefetch_idx():
                pltpu.async_copy(idx_hbm.at[wid, j + 2], ibuf.at[b], sis[b])
        return 0

    lax.fori_loop(0, CH // 2, body, 0)
    plsc.subcore_barrier()

    for off, n in ((0, C), (C, C), (2 * C, RPT - 2 * C)):
        pltpu.sync_copy(acc_sp.at[pl.ds(base + off, n)],
                        rows_v.at[0, pl.ds(0, n)])

        def sbody(r, _):
            r2 = off + r
            s = 1.0 / (degv[r2 // 8, pl.ds((r2 % 8) * 16, 16)] + EPS)
            for c in range(D // 16):
                rows_v[0, r, pl.ds(c * 16, 16)] = (
                    rows_v[0, r, pl.ds(c * 16, 16)] * s)
            return 0
        lax.fori_loop(0, n, sbody, 0)
        pltpu.sync_copy(rows_v.at[0, pl.ds(0, n)],
                        out_hbm.at[pl.ds(dst_off + off, n)])


_BR = 1280
_row_spec = pl.BlockSpec((_BR, D), lambda i: (i, 0))
_deg_spec = pl.BlockSpec((_BR, 16), lambda i: (i, 0))


def _deg_col(deg_ref):
    return deg_ref[:, 0:1] + EPS


def _tc_scale_body(x_ref, deg_ref, o_ref):
    o_ref[...] = x_ref[...] * lax.rsqrt(_deg_col(deg_ref))


_tc_scale = pl.pallas_call(
    _tc_scale_body,
    grid=(NT2 // _BR,),
    in_specs=[_row_spec, _deg_spec],
    out_specs=_row_spec,
    out_shape=jax.ShapeDtypeStruct((NT2, D), jnp.float32),
)


def _tc_final_body(z0_ref, z1_ref, z2_ref, z3_ref, deg_ref, o_ref):
    o_ref[...] = ((z0_ref[...] + z1_ref[...] + z2_ref[...] + z3_ref[...])
                  * (0.25 * lax.sqrt(_deg_col(deg_ref))))


_tc_final = pl.pallas_call(
    _tc_final_body,
    grid=(NT2 // _BR,),
    in_specs=[_row_spec, _row_spec, _row_spec, _row_spec, _deg_spec],
    out_specs=_row_spec,
    out_shape=jax.ShapeDtypeStruct((NT2, D), jnp.float32),
)


def kernel(user_emb, item_emb, edge_index):
    src = edge_index[0]
    dst = edge_index[1]

    srcH = jnp.concatenate([src[EH:] - NH, src[:EH]]).reshape(NW, EPW)
    dstH = jnp.concatenate([dst[EH:], dst[:EH] - NH]).reshape(NW, EPW)
    pad = CH * C - EPW
    src2 = jnp.pad(srcH, ((0, 0), (0, pad)), constant_values=0)
    dst2 = jnp.pad(dstH, ((0, 0), (0, pad)), constant_values=DROW)
    idx = jnp.stack([src2.reshape(NW, CH, 128),
                     dst2.reshape(NW, CH, 128)], axis=2)

    deg16 = _k0_degrees(idx).reshape(NT2, 16)

    padrows = jnp.zeros((NHP - NH, D), jnp.float32)
    all_emb = jnp.concatenate([user_emb, padrows, item_emb, padrows], axis=0)
    z0 = _tc_scale(all_emb, deg16)

    deg2d = deg16.reshape(NT2 // 8, 8 * 16)
    zs = [z0]
    for _ in range(N_LAYERS):
        zs.append(_k1_propagate(zs[-1], idx, deg2d))

    out = _tc_final(zs[0], zs[1], zs[2], zs[3], deg16)

    return (out[:N_USERS], out[NHP:NHP + N_ITEMS])


# speedup vs baseline: 2.1843x; 1.0279x over previous
import functools

import jax
import jax.numpy as jnp
from jax import lax
from jax.experimental import pallas as pl
from jax.experimental.pallas import tpu as pltpu
from jax.experimental.pallas import tpu_sc as plsc

N_USERS = 5000
N_ITEMS = 5000
NT = N_USERS + N_ITEMS
NH = NT // 2
NHP = 5120
NT2 = 2 * NHP
D = 128
E = 320000
EH = E // 2
N_LAYERS = 3
EPS = 1e-7

NC = 2
NS = 16
NW = NC * NS

C = 128
EPW = EH // NS
CH = 80
RPT = NHP // NS
DROW = NH

_mesh = plsc.VectorSubcoreMesh(core_axis_name="c", subcore_axis_name="s")


@functools.partial(
    pl.kernel,
    out_type=jax.ShapeDtypeStruct((NC, NS * RPT, 16), jnp.float32),
    mesh=_mesh,
    scratch_types=[
        pltpu.VMEM((2, 2, 2, 128), jnp.int32),
        pltpu.VMEM((128, 16), jnp.float32),
        pltpu.VMEM((128, 16), jnp.float32),
        pltpu.VMEM_SHARED((NS * RPT, 16), jnp.float32),
        pltpu.SemaphoreType.DMA,
        pltpu.SemaphoreType.DMA,
    ],
)
def _k0_degrees(idx_hbm, deg_hbm, ibuf, ones_v, zrow_v, deg_sp, si0, si1):
    cid = lax.axis_index("c")
    sid = lax.axis_index("s")
    wid = cid * NS + sid
    sis = (si0, si1)

    one_row = jnp.full((16,), 1.0, jnp.float32)
    zrow = jnp.zeros((16,), jnp.float32)

    def init_rows(i, _):
        ones_v[i, :] = one_row
        zrow_v[i, :] = zrow
        return 0
    lax.fori_loop(0, 128, init_rows, 0)

    base = sid * RPT
    for k in range(RPT // 128):
        pltpu.sync_copy(zrow_v, deg_sp.at[pl.ds(base + k * 128, 128)])
    pltpu.sync_copy(zrow_v.at[pl.ds(0, 64)],
                    deg_sp.at[pl.ds(base + 256, 64)])
    plsc.subcore_barrier()

    pltpu.sync_copy(idx_hbm.at[wid, pl.ds(0, 2)], ibuf.at[0])

    def body(jj, _):
        for b in range(2):
            j = jj * 2 + b

            @pl.when(j > 0)
            def _wait():
                pltpu.make_async_copy(
                    idx_hbm.at[wid, pl.ds(j * 2, 2)], ibuf.at[b],
                    sis[b]).wait()

            @pl.when(j + 1 < CH // 2)
            def _issue():
                pltpu.async_copy(
                    idx_hbm.at[wid, pl.ds((j + 1) * 2, 2)], ibuf.at[1 - b],
                    sis[1 - b])

            for k in range(2):
                pltpu.sync_copy(ones_v, deg_sp.at[ibuf.at[b, k, 1]],
                                add=True)
        return 0

    lax.fori_loop(0, CH // 4, body, 0)
    plsc.subcore_barrier()

    out_base = pl.multiple_of(sid * RPT, 8)
    pltpu.sync_copy(deg_sp.at[pl.ds(out_base, RPT)],
                    deg_hbm.at[cid, pl.ds(out_base, RPT)])


@functools.partial(
    pl.kernel,
    out_type=jax.ShapeDtypeStruct((NT2, D), jnp.float32),
    mesh=_mesh,
    scratch_types=[
        pltpu.VMEM((2, 2, 128), jnp.int32),
        pltpu.VMEM((2, C, D), jnp.float32),
        pltpu.VMEM((RPT // 8, 128), jnp.float32),
        pltpu.VMEM_SHARED((NHP, D), jnp.float32),
        pltpu.VMEM_SHARED((NHP, D), jnp.float32),
        pltpu.SemaphoreType.DMA,
        pltpu.SemaphoreType.DMA,
        pltpu.SemaphoreType.DMA,
        pltpu.SemaphoreType.DMA,
    ],
)
def _k1_propagate(z_hbm, idx_hbm, deg_hbm, out_hbm, ibuf, rows_v, degv,
                  zsp, acc_sp, si0, si1, sg0, sg1):
    cid = lax.axis_index("c")
    sid = lax.axis_index("s")
    wid = cid * NS + sid
    sis = (si0, si1)
    sgs = (sg0, sg1)

    base = pl.multiple_of(sid * RPT, 8)

    src_off = pl.multiple_of((1 - cid) * NHP + base, 8)
    pltpu.sync_copy(z_hbm.at[pl.ds(src_off, RPT)], zsp.at[pl.ds(base, RPT)])
    dst_off = pl.multiple_of(cid * NHP + base, 8)
    pltpu.sync_copy(
        deg_hbm.at[pl.ds(pl.multiple_of(dst_off // 8, 8), RPT // 8)], degv)

    zv = jnp.zeros((16,), jnp.float32)

    def zbody(i, _):
        r = i // (D // 16)
        col = (i % (D // 16)) * 16
        rows_v[0, r, pl.ds(col, 16)] = zv
        return 0
    lax.fori_loop(0, C * (D // 16), zbody, 0)

    for k in range(RPT // C):
        pltpu.sync_copy(rows_v.at[0], acc_sp.at[pl.ds(base + k * C, C)])
    pltpu.sync_copy(rows_v.at[0, pl.ds(0, RPT % C)],
                    acc_sp.at[pl.ds(base + (RPT // C) * C, RPT % C)])
    plsc.subcore_barrier()

    pltpu.sync_copy(idx_hbm.at[wid, 0], ibuf.at[0])
    pltpu.async_copy(zsp.at[ibuf.at[0, 0]], rows_v.at[0], sg0)
    pltpu.async_copy(idx_hbm.at[wid, 1], ibuf.at[1], si1)

    def body(jj, _):
        for b in range(2):
            j = jj * 2 + b

            @pl.when(j + 1 < CH)
            def _gather_next():
                pltpu.make_async_copy(
                    idx_hbm.at[wid, j + 1], ibuf.at[1 - b], sis[1 - b]).wait()
                pltpu.async_copy(
                    zsp.at[ibuf.at[1 - b, 0]], rows_v.at[1 - b], sgs[1 - b])

            pltpu.make_async_copy(
                zsp.at[ibuf.at[b, 0]], rows_v.at[b], sgs[b]).wait()
            pltpu.sync_copy(rows_v.at[b], acc_sp.at[ibuf.at[b, 1]], add=True)

            @pl.when(j + 2 < CH)
            def _prefetch_idx():
                pltpu.async_copy(idx_hbm.at[wid, j + 2], ibuf.at[b], sis[b])
        return 0

    lax.fori_loop(0, CH // 2, body, 0)
    plsc.subcore_barrier()

    for off, n in ((0, C), (C, C), (2 * C, RPT - 2 * C)):
        pltpu.sync_copy(acc_sp.at[pl.ds(base + off, n)],
                        rows_v.at[0, pl.ds(0, n)])

        def sbody(r, _):
            r2 = off + r
            s = 1.0 / (degv[r2 // 8, pl.ds((r2 % 8) * 16, 16)] + EPS)
            for c in range(D // 16):
                rows_v[0, r, pl.ds(c * 16, 16)] = (
                    rows_v[0, r, pl.ds(c * 16, 16)] * s)
            return 0
        lax.fori_loop(0, n, sbody, 0)
        pltpu.sync_copy(rows_v.at[0, pl.ds(0, n)],
                        out_hbm.at[pl.ds(dst_off + off, n)])


_BR = 1280
_row_spec = pl.BlockSpec((_BR, D), lambda i: (i, 0))
_deg_spec = pl.BlockSpec((_BR, 16), lambda i: (i, 0))


def _deg_col(deg_ref):
    return deg_ref[:, 0:1] + EPS


def _tc_scale_body(x_ref, deg_ref, o_ref):
    o_ref[...] = x_ref[...] * lax.rsqrt(_deg_col(deg_ref))


_tc_scale = pl.pallas_call(
    _tc_scale_body,
    grid=(NT2 // _BR,),
    in_specs=[_row_spec, _deg_spec],
    out_specs=_row_spec,
    out_shape=jax.ShapeDtypeStruct((NT2, D), jnp.float32),
)


def _tc_final_body(z0_ref, z1_ref, z2_ref, z3_ref, deg_ref, o_ref):
    o_ref[...] = ((z0_ref[...] + z1_ref[...] + z2_ref[...] + z3_ref[...])
                  * (0.25 * lax.sqrt(_deg_col(deg_ref))))


_tc_final = pl.pallas_call(
    _tc_final_body,
    grid=(NT2 // _BR,),
    in_specs=[_row_spec, _row_spec, _row_spec, _row_spec, _deg_spec],
    out_specs=_row_spec,
    out_shape=jax.ShapeDtypeStruct((NT2, D), jnp.float32),
)


def kernel(user_emb, item_emb, edge_index):
    src = edge_index[0]
    dst = edge_index[1]

    srcH = jnp.concatenate([src[EH:] - NH, src[:EH]]).reshape(NW, EPW)
    dstH = jnp.concatenate([dst[EH:], dst[:EH] - NH]).reshape(NW, EPW)
    pad = CH * C - EPW
    src2 = jnp.pad(srcH, ((0, 0), (0, pad)), constant_values=0)
    dst2 = jnp.pad(dstH, ((0, 0), (0, pad)), constant_values=DROW)
    idx = jnp.stack([src2.reshape(NW, CH, 128),
                     dst2.reshape(NW, CH, 128)], axis=2)

    deg16 = _k0_degrees(idx).reshape(NT2, 16)

    padrows = jnp.zeros((NHP - NH, D), jnp.float32)
    all_emb = jnp.concatenate([user_emb, padrows, item_emb, padrows], axis=0)
    z0 = _tc_scale(all_emb, deg16)

    deg2d = deg16.reshape(NT2 // 8, 8 * 16)
    zs = [z0]
    for _ in range(N_LAYERS):
        zs.append(_k1_propagate(zs[-1], idx, deg2d))

    out = _tc_final(zs[0], zs[1], zs[2], zs[3], deg16)

    return (out[:N_USERS], out[NHP:NHP + N_ITEMS])


# speedup vs baseline: 2.4408x; 1.1174x over previous
import functools

import jax
import jax.numpy as jnp
from jax import lax
from jax.experimental import pallas as pl
from jax.experimental.pallas import tpu as pltpu
from jax.experimental.pallas import tpu_sc as plsc

N_USERS = 5000
N_ITEMS = 5000
NT = N_USERS + N_ITEMS
NH = NT // 2
NHP = 5120
NT2 = 2 * NHP
D = 128
E = 320000
EH = E // 2
N_LAYERS = 3
EPS = 1e-7

NC = 2
NS = 16
NW = NC * NS

C = 128
EPW = EH // NS
CH = 80
RPT = NHP // NS
DROW = NH

_mesh = plsc.VectorSubcoreMesh(core_axis_name="c", subcore_axis_name="s")


@functools.partial(
    pl.kernel,
    out_type=jax.ShapeDtypeStruct((NC, NS * RPT, 16), jnp.float32),
    mesh=_mesh,
    scratch_types=[
        pltpu.VMEM((2, 2, 2, 128), jnp.int32),
        pltpu.VMEM((128, 16), jnp.float32),
        pltpu.VMEM((128, 16), jnp.float32),
        pltpu.VMEM_SHARED((NS * RPT, 16), jnp.float32),
        pltpu.SemaphoreType.DMA,
        pltpu.SemaphoreType.DMA,
    ],
)
def _k0_degrees(idx_hbm, deg_hbm, ibuf, ones_v, zrow_v, deg_sp, si0, si1):
    cid = lax.axis_index("c")
    sid = lax.axis_index("s")
    wid = cid * NS + sid
    sis = (si0, si1)

    one_row = jnp.full((16,), 1.0, jnp.float32)
    zrow = jnp.zeros((16,), jnp.float32)

    def init_rows(i, _):
        ones_v[i, :] = one_row
        zrow_v[i, :] = zrow
        return 0
    lax.fori_loop(0, 128, init_rows, 0)

    base = sid * RPT
    for k in range(RPT // 128):
        pltpu.sync_copy(zrow_v, deg_sp.at[pl.ds(base + k * 128, 128)])
    pltpu.sync_copy(zrow_v.at[pl.ds(0, 64)],
                    deg_sp.at[pl.ds(base + 256, 64)])
    plsc.subcore_barrier()

    pltpu.sync_copy(idx_hbm.at[wid, pl.ds(0, 2)], ibuf.at[0])

    def body(jj, _):
        for b in range(2):
            j = jj * 2 + b

            @pl.when(j > 0)
            def _wait():
                pltpu.make_async_copy(
                    idx_hbm.at[wid, pl.ds(j * 2, 2)], ibuf.at[b],
                    sis[b]).wait()

            @pl.when(j + 1 < CH // 2)
            def _issue():
                pltpu.async_copy(
                    idx_hbm.at[wid, pl.ds((j + 1) * 2, 2)], ibuf.at[1 - b],
                    sis[1 - b])

            for k in range(2):
                pltpu.sync_copy(ones_v, deg_sp.at[ibuf.at[b, k, 1]],
                                add=True)
        return 0

    lax.fori_loop(0, CH // 4, body, 0)
    plsc.subcore_barrier()

    out_base = pl.multiple_of(sid * RPT, 8)
    pltpu.sync_copy(deg_sp.at[pl.ds(out_base, RPT)],
                    deg_hbm.at[cid, pl.ds(out_base, RPT)])


@functools.partial(
    pl.kernel,
    out_type=jax.ShapeDtypeStruct((NT2, D), jnp.float32),
    mesh=_mesh,
    scratch_types=[
        pltpu.VMEM((3, 2, 128), jnp.int32),
        pltpu.VMEM((2, C, D), jnp.float32),
        pltpu.VMEM((RPT // 8, 128), jnp.float32),
        pltpu.VMEM_SHARED((NHP, D), jnp.float32),
        pltpu.VMEM_SHARED((NHP, D), jnp.float32),
        pltpu.SemaphoreType.DMA,
        pltpu.SemaphoreType.DMA,
        pltpu.SemaphoreType.DMA,
        pltpu.SemaphoreType.DMA,
        pltpu.SemaphoreType.DMA,
        pltpu.SemaphoreType.DMA,
    ],
)
def _k1_propagate(z_hbm, idx_hbm, deg_hbm, out_hbm, ibuf, rows_v, degv,
                  zsp, acc_sp, si0, si1, sg0, sg1, ss0, ss1):
    cid = lax.axis_index("c")
    sid = lax.axis_index("s")
    wid = cid * NS + sid
    sis = (si0, si1)
    sgs = (sg0, sg1)
    sss = (ss0, ss1)

    base = pl.multiple_of(sid * RPT, 8)

    src_off = pl.multiple_of((1 - cid) * NHP + base, 8)
    pltpu.sync_copy(z_hbm.at[pl.ds(src_off, RPT)], zsp.at[pl.ds(base, RPT)])
    dst_off = pl.multiple_of(cid * NHP + base, 8)
    pltpu.sync_copy(
        deg_hbm.at[pl.ds(pl.multiple_of(dst_off // 8, 8), RPT // 8)], degv)

    zv = jnp.zeros((16,), jnp.float32)

    def zbody(i, _):
        r = i // (D // 16)
        col = (i % (D // 16)) * 16
        rows_v[0, r, pl.ds(col, 16)] = zv
        return 0
    lax.fori_loop(0, C * (D // 16), zbody, 0)

    for k in range(RPT // C):
        pltpu.sync_copy(rows_v.at[0], acc_sp.at[pl.ds(base + k * C, C)])
    pltpu.sync_copy(rows_v.at[0, pl.ds(0, RPT % C)],
                    acc_sp.at[pl.ds(base + (RPT // C) * C, RPT % C)])
    plsc.subcore_barrier()

    pltpu.sync_copy(idx_hbm.at[wid, 0], ibuf.at[0])
    pltpu.async_copy(zsp.at[ibuf.at[0, 0]], rows_v.at[0], sg0)
    pltpu.async_copy(idx_hbm.at[wid, 1], ibuf.at[1], si1)

    def body(jj, _):
        for b in range(2):
            j = jj * 2 + b

            @pl.when(j > 0)
            def _drain_scatter():
                pltpu.make_async_copy(
                    rows_v.at[1 - b], acc_sp.at[ibuf.at[(j - 1) % 3, 1]],
                    sss[1 - b]).wait()

            @pl.when(j + 1 < CH)
            def _gather_next():
                pltpu.make_async_copy(
                    idx_hbm.at[wid, j + 1], ibuf.at[(j + 1) % 3],
                    sis[1 - b]).wait()
                pltpu.async_copy(
                    zsp.at[ibuf.at[(j + 1) % 3, 0]], rows_v.at[1 - b],
                    sgs[1 - b])

                @pl.when(j + 2 < CH)
                def _prefetch_idx():
                    pltpu.async_copy(idx_hbm.at[wid, j + 2],
                                     ibuf.at[(j + 2) % 3], sis[b])

            pltpu.make_async_copy(
                zsp.at[ibuf.at[j % 3, 0]], rows_v.at[b], sgs[b]).wait()
            pltpu.async_copy(rows_v.at[b], acc_sp.at[ibuf.at[j % 3, 1]],
                             sss[b], add=True)
        return 0

    lax.fori_loop(0, CH // 2, body, 0)
    pltpu.make_async_copy(
        rows_v.at[1], acc_sp.at[ibuf.at[(CH - 1) % 3, 1]], sss[1]).wait()
    plsc.subcore_barrier()

    for off, n in ((0, C), (C, C), (2 * C, RPT - 2 * C)):
        pltpu.sync_copy(acc_sp.at[pl.ds(base + off, n)],
                        rows_v.at[0, pl.ds(0, n)])

        def sbody(r, _):
            r2 = off + r
            s = 1.0 / (degv[r2 // 8, pl.ds((r2 % 8) * 16, 16)] + EPS)
            for c in range(D // 16):
                rows_v[0, r, pl.ds(c * 16, 16)] = (
                    rows_v[0, r, pl.ds(c * 16, 16)] * s)
            return 0
        lax.fori_loop(0, n, sbody, 0)
        pltpu.sync_copy(rows_v.at[0, pl.ds(0, n)],
                        out_hbm.at[pl.ds(dst_off + off, n)])


_BR = 1280
_row_spec = pl.BlockSpec((_BR, D), lambda i: (i, 0))
_deg_spec = pl.BlockSpec((_BR, 16), lambda i: (i, 0))


def _deg_col(deg_ref):
    return deg_ref[:, 0:1] + EPS


def _tc_scale_body(x_ref, deg_ref, o_ref):
    o_ref[...] = x_ref[...] * lax.rsqrt(_deg_col(deg_ref))


_tc_scale = pl.pallas_call(
    _tc_scale_body,
    grid=(NT2 // _BR,),
    in_specs=[_row_spec, _deg_spec],
    out_specs=_row_spec,
    out_shape=jax.ShapeDtypeStruct((NT2, D), jnp.float32),
)


def _tc_final_body(z0_ref, z1_ref, z2_ref, z3_ref, deg_ref, o_ref):
    o_ref[...] = ((z0_ref[...] + z1_ref[...] + z2_ref[...] + z3_ref[...])
                  * (0.25 * lax.sqrt(_deg_col(deg_ref))))


_tc_final = pl.pallas_call(
    _tc_final_body,
    grid=(NT2 // _BR,),
    in_specs=[_row_spec, _row_spec, _row_spec, _row_spec, _deg_spec],
    out_specs=_row_spec,
    out_shape=jax.ShapeDtypeStruct((NT2, D), jnp.float32),
)


def kernel(user_emb, item_emb, edge_index):
    src = edge_index[0]
    dst = edge_index[1]

    srcH = jnp.concatenate([src[EH:] - NH, src[:EH]]).reshape(NW, EPW)
    dstH = jnp.concatenate([dst[EH:], dst[:EH] - NH]).reshape(NW, EPW)
    pad = CH * C - EPW
    src2 = jnp.pad(srcH, ((0, 0), (0, pad)), constant_values=0)
    dst2 = jnp.pad(dstH, ((0, 0), (0, pad)), constant_values=DROW)
    idx = jnp.stack([src2.reshape(NW, CH, 128),
                     dst2.reshape(NW, CH, 128)], axis=2)

    deg16 = _k0_degrees(idx).reshape(NT2, 16)

    padrows = jnp.zeros((NHP - NH, D), jnp.float32)
    all_emb = jnp.concatenate([user_emb, padrows, item_emb, padrows], axis=0)
    z0 = _tc_scale(all_emb, deg16)

    deg2d = deg16.reshape(NT2 // 8, 8 * 16)
    zs = [z0]
    for _ in range(N_LAYERS):
        zs.append(_k1_propagate(zs[-1], idx, deg2d))

    out = _tc_final(zs[0], zs[1], zs[2], zs[3], deg16)

    return (out[:N_USERS], out[NHP:NHP + N_ITEMS])


# speedup vs baseline: 2.4490x; 1.0034x over previous
import functools

import jax
import jax.numpy as jnp
from jax import lax
from jax.experimental import pallas as pl
from jax.experimental.pallas import tpu as pltpu
from jax.experimental.pallas import tpu_sc as plsc

N_USERS = 5000
N_ITEMS = 5000
NT = N_USERS + N_ITEMS
NH = NT // 2
NHP = 5120
NT2 = 2 * NHP
D = 128
E = 320000
EH = E // 2
N_LAYERS = 3
EPS = 1e-7

NC = 2
NS = 16
NW = NC * NS

C = 128
EPW = EH // NS
CH = 80
RPT = NHP // NS
DROW = NH

_mesh = plsc.VectorSubcoreMesh(core_axis_name="c", subcore_axis_name="s")


@functools.partial(
    pl.kernel,
    out_type=jax.ShapeDtypeStruct((NC, NS * RPT, 16), jnp.float32),
    mesh=_mesh,
    scratch_types=[
        pltpu.VMEM((2, 2, 2, 128), jnp.int32),
        pltpu.VMEM((128, 16), jnp.float32),
        pltpu.VMEM((128, 16), jnp.float32),
        pltpu.VMEM_SHARED((NS * RPT, 16), jnp.float32),
        pltpu.SemaphoreType.DMA,
        pltpu.SemaphoreType.DMA,
        pltpu.SemaphoreType.DMA,
        pltpu.SemaphoreType.DMA,
    ],
)
def _k0_degrees(idx_hbm, deg_hbm, ibuf, ones_v, zrow_v, deg_sp,
                si0, si1, ss0, ss1):
    cid = lax.axis_index("c")
    sid = lax.axis_index("s")
    wid = cid * NS + sid
    sis = (si0, si1)
    sss = (ss0, ss1)

    one_row = jnp.full((16,), 1.0, jnp.float32)
    zrow = jnp.zeros((16,), jnp.float32)

    def init_rows(i, _):
        ones_v[i, :] = one_row
        zrow_v[i, :] = zrow
        return 0
    lax.fori_loop(0, 128, init_rows, 0)

    base = sid * RPT
    for k in range(RPT // 128):
        pltpu.sync_copy(zrow_v, deg_sp.at[pl.ds(base + k * 128, 128)])
    pltpu.sync_copy(zrow_v.at[pl.ds(0, 64)],
                    deg_sp.at[pl.ds(base + 256, 64)])
    plsc.subcore_barrier()

    pltpu.sync_copy(idx_hbm.at[wid, pl.ds(0, 2)], ibuf.at[0])

    def body(jj, _):
        for b in range(2):
            j = jj * 2 + b

            @pl.when(j > 0)
            def _wait():
                pltpu.make_async_copy(
                    idx_hbm.at[wid, pl.ds(j * 2, 2)], ibuf.at[b],
                    sis[b]).wait()
                for k in range(2):
                    pltpu.make_async_copy(
                        ones_v, deg_sp.at[ibuf.at[1 - b, k, 1]],
                        sss[1 - b]).wait()

            @pl.when(j + 1 < CH // 2)
            def _issue():
                pltpu.async_copy(
                    idx_hbm.at[wid, pl.ds((j + 1) * 2, 2)], ibuf.at[1 - b],
                    sis[1 - b])

            for k in range(2):
                pltpu.async_copy(ones_v, deg_sp.at[ibuf.at[b, k, 1]],
                                 sss[b], add=True)
        return 0

    lax.fori_loop(0, CH // 4, body, 0)
    for k in range(2):
        pltpu.make_async_copy(
            ones_v, deg_sp.at[ibuf.at[1, k, 1]], sss[1]).wait()
    plsc.subcore_barrier()

    out_base = pl.multiple_of(sid * RPT, 8)
    pltpu.sync_copy(deg_sp.at[pl.ds(out_base, RPT)],
                    deg_hbm.at[cid, pl.ds(out_base, RPT)])


@functools.partial(
    pl.kernel,
    out_type=jax.ShapeDtypeStruct((NT2, D), jnp.float32),
    mesh=_mesh,
    scratch_types=[
        pltpu.VMEM((3, 2, 128), jnp.int32),
        pltpu.VMEM((2, C, D), jnp.float32),
        pltpu.VMEM((RPT // 8, 128), jnp.float32),
        pltpu.VMEM_SHARED((NHP, D), jnp.float32),
        pltpu.VMEM_SHARED((NHP, D), jnp.float32),
        pltpu.SemaphoreType.DMA,
        pltpu.SemaphoreType.DMA,
        pltpu.SemaphoreType.DMA,
        pltpu.SemaphoreType.DMA,
        pltpu.SemaphoreType.DMA,
        pltpu.SemaphoreType.DMA,
    ],
)
def _k1_propagate(z_hbm, idx_hbm, deg_hbm, out_hbm, ibuf, rows_v, degv,
                  zsp, acc_sp, si0, si1, sg0, sg1, ss0, ss1):
    cid = lax.axis_index("c")
    sid = lax.axis_index("s")
    wid = cid * NS + sid
    sis = (si0, si1)
    sgs = (sg0, sg1)
    sss = (ss0, ss1)

    base = pl.multiple_of(sid * RPT, 8)

    src_off = pl.multiple_of((1 - cid) * NHP + base, 8)
    pltpu.sync_copy(z_hbm.at[pl.ds(src_off, RPT)], zsp.at[pl.ds(base, RPT)])
    dst_off = pl.multiple_of(cid * NHP + base, 8)
    pltpu.sync_copy(
        deg_hbm.at[pl.ds(pl.multiple_of(dst_off // 8, 8), RPT // 8)], degv)

    zv = jnp.zeros((16,), jnp.float32)

    def zbody(i, _):
        r = i // (D // 16)
        col = (i % (D // 16)) * 16
        rows_v[0, r, pl.ds(col, 16)] = zv
        return 0
    lax.fori_loop(0, C * (D // 16), zbody, 0)

    for k in range(RPT // C):
        pltpu.sync_copy(rows_v.at[0], acc_sp.at[pl.ds(base + k * C, C)])
    pltpu.sync_copy(rows_v.at[0, pl.ds(0, RPT % C)],
                    acc_sp.at[pl.ds(base + (RPT // C) * C, RPT % C)])
    plsc.subcore_barrier()

    pltpu.sync_copy(idx_hbm.at[wid, 0], ibuf.at[0])
    pltpu.async_copy(zsp.at[ibuf.at[0, 0]], rows_v.at[0], sg0)
    pltpu.async_copy(idx_hbm.at[wid, 1], ibuf.at[1], si1)

    def body(jj, _):
        for b in range(2):
            j = jj * 2 + b

            @pl.when(j > 0)
            def _drain_scatter():
                pltpu.make_async_copy(
                    rows_v.at[1 - b], acc_sp.at[ibuf.at[(j - 1) % 3, 1]],
                    sss[1 - b]).wait()

            @pl.when(j + 1 < CH)
            def _gather_next():
                pltpu.make_async_copy(
                    idx_hbm.at[wid, j + 1], ibuf.at[(j + 1) % 3],
                    sis[1 - b]).wait()
                pltpu.async_copy(
                    zsp.at[ibuf.at[(j + 1) % 3, 0]], rows_v.at[1 - b],
                    sgs[1 - b])

                @pl.when(j + 2 < CH)
                def _prefetch_idx():
                    pltpu.async_copy(idx_hbm.at[wid, j + 2],
                                     ibuf.at[(j + 2) % 3], sis[b])

            pltpu.make_async_copy(
                zsp.at[ibuf.at[j % 3, 0]], rows_v.at[b], sgs[b]).wait()
            pltpu.async_copy(rows_v.at[b], acc_sp.at[ibuf.at[j % 3, 1]],
                             sss[b], add=True)
        return 0

    lax.fori_loop(0, CH // 2, body, 0)
    pltpu.make_async_copy(
        rows_v.at[1], acc_sp.at[ibuf.at[(CH - 1) % 3, 1]], sss[1]).wait()
    plsc.subcore_barrier()

    for off, n in ((0, C), (C, C), (2 * C, RPT - 2 * C)):
        pltpu.sync_copy(acc_sp.at[pl.ds(base + off, n)],
                        rows_v.at[0, pl.ds(0, n)])

        def sbody(r, _):
            r2 = off + r
            s = 1.0 / (degv[r2 // 8, pl.ds((r2 % 8) * 16, 16)] + EPS)
            for c in range(D // 16):
                rows_v[0, r, pl.ds(c * 16, 16)] = (
                    rows_v[0, r, pl.ds(c * 16, 16)] * s)
            return 0
        lax.fori_loop(0, n, sbody, 0)
        pltpu.sync_copy(rows_v.at[0, pl.ds(0, n)],
                        out_hbm.at[pl.ds(dst_off + off, n)])


_BR = 1280
_row_spec = pl.BlockSpec((_BR, D), lambda i: (i, 0))
_deg_spec = pl.BlockSpec((_BR, 16), lambda i: (i, 0))


def _deg_col(deg_ref):
    return deg_ref[:, 0:1] + EPS


def _tc_scale_body(x_ref, deg_ref, o_ref):
    o_ref[...] = x_ref[...] * lax.rsqrt(_deg_col(deg_ref))


_tc_scale = pl.pallas_call(
    _tc_scale_body,
    grid=(NT2 // _BR,),
    in_specs=[_row_spec, _deg_spec],
    out_specs=_row_spec,
    out_shape=jax.ShapeDtypeStruct((NT2, D), jnp.float32),
)


def _tc_final_body(z0_ref, z1_ref, z2_ref, z3_ref, deg_ref, o_ref):
    o_ref[...] = ((z0_ref[...] + z1_ref[...] + z2_ref[...] + z3_ref[...])
                  * (0.25 * lax.sqrt(_deg_col(deg_ref))))


_tc_final = pl.pallas_call(
    _tc_final_body,
    grid=(NT2 // _BR,),
    in_specs=[_row_spec, _row_spec, _row_spec, _row_spec, _deg_spec],
    out_specs=_row_spec,
    out_shape=jax.ShapeDtypeStruct((NT2, D), jnp.float32),
)


def kernel(user_emb, item_emb, edge_index):
    src = edge_index[0]
    dst = edge_index[1]

    srcH = jnp.concatenate([src[EH:] - NH, src[:EH]]).reshape(NW, EPW)
    dstH = jnp.concatenate([dst[EH:], dst[:EH] - NH]).reshape(NW, EPW)
    pad = CH * C - EPW
    src2 = jnp.pad(srcH, ((0, 0), (0, pad)), constant_values=0)
    dst2 = jnp.pad(dstH, ((0, 0), (0, pad)), constant_values=DROW)
    idx = jnp.stack([src2.reshape(NW, CH, 128),
                     dst2.reshape(NW, CH, 128)], axis=2)

    deg16 = _k0_degrees(idx).reshape(NT2, 16)

    padrows = jnp.zeros((NHP - NH, D), jnp.float32)
    all_emb = jnp.concatenate([user_emb, padrows, item_emb, padrows], axis=0)
    z0 = _tc_scale(all_emb, deg16)

    deg2d = deg16.reshape(NT2 // 8, 8 * 16)
    zs = [z0]
    for _ in range(N_LAYERS):
        zs.append(_k1_propagate(zs[-1], idx, deg2d))

    out = _tc_final(zs[0], zs[1], zs[2], zs[3], deg16)

    return (out[:N_USERS], out[NHP:NHP + N_ITEMS])


# speedup vs baseline: 2.4994x; 1.0206x over previous
import functools

import jax
import jax.numpy as jnp
from jax import lax
from jax.experimental import pallas as pl
from jax.experimental.pallas import tpu as pltpu
from jax.experimental.pallas import tpu_sc as plsc

N_USERS = 5000
N_ITEMS = 5000
NT = N_USERS + N_ITEMS
NH = NT // 2
NHP = 5120
NT2 = 2 * NHP
D = 128
E = 320000
EH = E // 2
N_LAYERS = 3
EPS = 1e-7

NC = 2
NS = 16
NW = NC * NS

C = 128
EPW = EH // NS
CH = 80
RPT = NHP // NS
DROW = NH

_mesh = plsc.VectorSubcoreMesh(core_axis_name="c", subcore_axis_name="s")


@functools.partial(
    pl.kernel,
    out_type=jax.ShapeDtypeStruct((NC, NS * RPT, 16), jnp.float32),
    mesh=_mesh,
    scratch_types=[
        pltpu.VMEM((2, 2, 2, 128), jnp.int32),
        pltpu.VMEM((128, 16), jnp.float32),
        pltpu.VMEM((128, 16), jnp.float32),
        pltpu.VMEM_SHARED((NS * RPT, 16), jnp.float32),
        pltpu.SemaphoreType.DMA,
        pltpu.SemaphoreType.DMA,
        pltpu.SemaphoreType.DMA,
        pltpu.SemaphoreType.DMA,
    ],
)
def _k0_degrees(idx_hbm, deg_hbm, ibuf, ones_v, zrow_v, deg_sp,
                si0, si1, ss0, ss1):
    cid = lax.axis_index("c")
    sid = lax.axis_index("s")
    wid = cid * NS + sid
    sis = (si0, si1)
    sss = (ss0, ss1)

    one_row = jnp.full((16,), 1.0, jnp.float32)
    zrow = jnp.zeros((16,), jnp.float32)

    def init_rows(i, _):
        ones_v[i, :] = one_row
        zrow_v[i, :] = zrow
        return 0
    lax.fori_loop(0, 128, init_rows, 0)

    base = sid * RPT
    for k in range(RPT // 128):
        pltpu.sync_copy(zrow_v, deg_sp.at[pl.ds(base + k * 128, 128)])
    pltpu.sync_copy(zrow_v.at[pl.ds(0, 64)],
                    deg_sp.at[pl.ds(base + 256, 64)])
    plsc.subcore_barrier()

    pltpu.sync_copy(idx_hbm.at[wid, pl.ds(0, 2)], ibuf.at[0])

    def body(jj, _):
        for b in range(2):
            j = jj * 2 + b

            @pl.when(j > 0)
            def _wait():
                pltpu.make_async_copy(
                    idx_hbm.at[wid, pl.ds(j * 2, 2)], ibuf.at[b],
                    sis[b]).wait()
                for k in range(2):
                    pltpu.make_async_copy(
                        ones_v, deg_sp.at[ibuf.at[1 - b, k, 1]],
                        sss[1 - b]).wait()

            @pl.when(j + 1 < CH // 2)
            def _issue():
                pltpu.async_copy(
                    idx_hbm.at[wid, pl.ds((j + 1) * 2, 2)], ibuf.at[1 - b],
                    sis[1 - b])

            for k in range(2):
                pltpu.async_copy(ones_v, deg_sp.at[ibuf.at[b, k, 1]],
                                 sss[b], add=True)
        return 0

    lax.fori_loop(0, CH // 4, body, 0)
    for k in range(2):
        pltpu.make_async_copy(
            ones_v, deg_sp.at[ibuf.at[1, k, 1]], sss[1]).wait()
    plsc.subcore_barrier()

    out_base = pl.multiple_of(sid * RPT, 8)
    pltpu.sync_copy(deg_sp.at[pl.ds(out_base, RPT)],
                    deg_hbm.at[cid, pl.ds(out_base, RPT)])


@functools.partial(
    pl.kernel,
    out_type=jax.ShapeDtypeStruct((NT2, D), jnp.float32),
    mesh=_mesh,
    scratch_types=[
        pltpu.VMEM((3, 2, 128), jnp.int32),
        pltpu.VMEM((2, C, D), jnp.float32),
        pltpu.VMEM((RPT // 8, 128), jnp.float32),
        pltpu.VMEM_SHARED((NHP, D), jnp.float32),
        pltpu.VMEM_SHARED((NHP, D), jnp.float32),
        pltpu.SemaphoreType.DMA,
        pltpu.SemaphoreType.DMA,
        pltpu.SemaphoreType.DMA,
        pltpu.SemaphoreType.DMA,
        pltpu.SemaphoreType.DMA,
        pltpu.SemaphoreType.DMA,
    ],
)
def _k1_propagate(z_hbm, idx_hbm, deg_hbm, out_hbm, ibuf, rows_v, degv,
                  zsp, acc_sp, si0, si1, sg0, sg1, ss0, ss1):
    cid = lax.axis_index("c")
    sid = lax.axis_index("s")
    wid = cid * NS + sid
    sis = (si0, si1)
    sgs = (sg0, sg1)
    sss = (ss0, ss1)

    base = pl.multiple_of(sid * RPT, 8)

    src_off = pl.multiple_of((1 - cid) * NHP + base, 8)
    stage_cp = pltpu.async_copy(
        z_hbm.at[pl.ds(src_off, RPT)], zsp.at[pl.ds(base, RPT)], sg0)
    dst_off = pl.multiple_of(cid * NHP + base, 8)
    deg_cp = pltpu.async_copy(
        deg_hbm.at[pl.ds(pl.multiple_of(dst_off // 8, 8), RPT // 8)], degv,
        sg1)

    zv = jnp.zeros((16,), jnp.float32)

    def zbody(i, _):
        r = i // (D // 16)
        col = (i % (D // 16)) * 16
        rows_v[0, r, pl.ds(col, 16)] = zv
        return 0
    lax.fori_loop(0, C * (D // 16), zbody, 0)

    for k in range(RPT // C):
        pltpu.sync_copy(rows_v.at[0], acc_sp.at[pl.ds(base + k * C, C)])
    pltpu.sync_copy(rows_v.at[0, pl.ds(0, RPT % C)],
                    acc_sp.at[pl.ds(base + (RPT // C) * C, RPT % C)])
    stage_cp.wait()
    deg_cp.wait()
    plsc.subcore_barrier()

    pltpu.sync_copy(idx_hbm.at[wid, 0], ibuf.at[0])
    pltpu.async_copy(zsp.at[ibuf.at[0, 0]], rows_v.at[0], sg0)
    pltpu.async_copy(idx_hbm.at[wid, 1], ibuf.at[1], si1)

    def body(jj, _):
        for b in range(2):
            j = jj * 2 + b

            @pl.when(j > 0)
            def _drain_scatter():
                pltpu.make_async_copy(
                    rows_v.at[1 - b], acc_sp.at[ibuf.at[(j - 1) % 3, 1]],
                    sss[1 - b]).wait()

            @pl.when(j + 1 < CH)
            def _gather_next():
                pltpu.make_async_copy(
                    idx_hbm.at[wid, j + 1], ibuf.at[(j + 1) % 3],
                    sis[1 - b]).wait()
                pltpu.async_copy(
                    zsp.at[ibuf.at[(j + 1) % 3, 0]], rows_v.at[1 - b],
                    sgs[1 - b])

                @pl.when(j + 2 < CH)
                def _prefetch_idx():
                    pltpu.async_copy(idx_hbm.at[wid, j + 2],
                                     ibuf.at[(j + 2) % 3], sis[b])

            pltpu.make_async_copy(
                zsp.at[ibuf.at[j % 3, 0]], rows_v.at[b], sgs[b]).wait()
            pltpu.async_copy(rows_v.at[b], acc_sp.at[ibuf.at[j % 3, 1]],
                             sss[b], add=True)
        return 0

    lax.fori_loop(0, CH // 2, body, 0)
    pltpu.make_async_copy(
        rows_v.at[1], acc_sp.at[ibuf.at[(CH - 1) % 3, 1]], sss[1]).wait()
    plsc.subcore_barrier()

    for off, n in ((0, C), (C, C), (2 * C, RPT - 2 * C)):
        pltpu.sync_copy(acc_sp.at[pl.ds(base + off, n)],
                        rows_v.at[0, pl.ds(0, n)])

        def sbody(r, _):
            r2 = off + r
            s = 1.0 / (degv[r2 // 8, pl.ds((r2 % 8) * 16, 16)] + EPS)
            for c in range(D // 16):
                rows_v[0, r, pl.ds(c * 16, 16)] = (
                    rows_v[0, r, pl.ds(c * 16, 16)] * s)
            return 0
        lax.fori_loop(0, n, sbody, 0)
        pltpu.sync_copy(rows_v.at[0, pl.ds(0, n)],
                        out_hbm.at[pl.ds(dst_off + off, n)])


_BR = 1280
_row_spec = pl.BlockSpec((_BR, D), lambda i: (i, 0))
_deg_spec = pl.BlockSpec((_BR, 16), lambda i: (i, 0))


def _deg_col(deg_ref):
    return deg_ref[:, 0:1] + EPS


def _tc_scale_body(x_ref, deg_ref, o_ref):
    o_ref[...] = x_ref[...] * lax.rsqrt(_deg_col(deg_ref))


_tc_scale = pl.pallas_call(
    _tc_scale_body,
    grid=(NT2 // _BR,),
    in_specs=[_row_spec, _deg_spec],
    out_specs=_row_spec,
    out_shape=jax.ShapeDtypeStruct((NT2, D), jnp.float32),
)


def _tc_final_body(z0_ref, z1_ref, z2_ref, z3_ref, deg_ref, o_ref):
    o_ref[...] = ((z0_ref[...] + z1_ref[...] + z2_ref[...] + z3_ref[...])
                  * (0.25 * lax.sqrt(_deg_col(deg_ref))))


_tc_final = pl.pallas_call(
    _tc_final_body,
    grid=(NT2 // _BR,),
    in_specs=[_row_spec, _row_spec, _row_spec, _row_spec, _deg_spec],
    out_specs=_row_spec,
    out_shape=jax.ShapeDtypeStruct((NT2, D), jnp.float32),
)


def kernel(user_emb, item_emb, edge_index):
    src = edge_index[0]
    dst = edge_index[1]

    srcH = jnp.concatenate([src[EH:] - NH, src[:EH]]).reshape(NW, EPW)
    dstH = jnp.concatenate([dst[EH:], dst[:EH] - NH]).reshape(NW, EPW)
    pad = CH * C - EPW
    src2 = jnp.pad(srcH, ((0, 0), (0, pad)), constant_values=0)
    dst2 = jnp.pad(dstH, ((0, 0), (0, pad)), constant_values=DROW)
    idx = jnp.stack([src2.reshape(NW, CH, 128),
                     dst2.reshape(NW, CH, 128)], axis=2)

    deg16 = _k0_degrees(idx).reshape(NT2, 16)

    padrows = jnp.zeros((NHP - NH, D), jnp.float32)
    all_emb = jnp.concatenate([user_emb, padrows, item_emb, padrows], axis=0)
    z0 = _tc_scale(all_emb, deg16)

    deg2d = deg16.reshape(NT2 // 8, 8 * 16)
    zs = [z0]
    for _ in range(N_LAYERS):
        zs.append(_k1_propagate(zs[-1], idx, deg2d))

    out = _tc_final(zs[0], zs[1], zs[2], zs[3], deg16)

    return (out[:N_USERS], out[NHP:NHP + N_ITEMS])


# speedup vs baseline: 2.5127x; 1.0053x over previous
import functools

import jax
import jax.numpy as jnp
from jax import lax
from jax.experimental import pallas as pl
from jax.experimental.pallas import tpu as pltpu
from jax.experimental.pallas import tpu_sc as plsc

N_USERS = 5000
N_ITEMS = 5000
NT = N_USERS + N_ITEMS
NH = NT // 2
NHP = 5120
NT2 = 2 * NHP
D = 128
E = 320000
EH = E // 2
N_LAYERS = 3
EPS = 1e-7

NC = 2
NS = 16
NW = NC * NS

C = 128
EPW = EH // NS
CH = 80
RPT = NHP // NS
DROW = NH

_mesh = plsc.VectorSubcoreMesh(core_axis_name="c", subcore_axis_name="s")


@functools.partial(
    pl.kernel,
    out_type=jax.ShapeDtypeStruct((NC, NS * RPT, 16), jnp.float32),
    mesh=_mesh,
    scratch_types=[
        pltpu.VMEM((2, 2, 2, 128), jnp.int32),
        pltpu.VMEM((128, 16), jnp.float32),
        pltpu.VMEM((128, 16), jnp.float32),
        pltpu.VMEM_SHARED((NS * RPT, 16), jnp.float32),
        pltpu.SemaphoreType.DMA,
        pltpu.SemaphoreType.DMA,
        pltpu.SemaphoreType.DMA,
        pltpu.SemaphoreType.DMA,
    ],
)
def _k0_degrees(idx_hbm, deg_hbm, ibuf, ones_v, zrow_v, deg_sp,
                si0, si1, ss0, ss1):
    cid = lax.axis_index("c")
    sid = lax.axis_index("s")
    wid = cid * NS + sid
    sis = (si0, si1)
    sss = (ss0, ss1)

    one_row = jnp.full((16,), 1.0, jnp.float32)
    zrow = jnp.zeros((16,), jnp.float32)

    def init_rows(i, _):
        ones_v[i, :] = one_row
        zrow_v[i, :] = zrow
        return 0
    lax.fori_loop(0, 128, init_rows, 0)

    base = sid * RPT
    for k in range(RPT // 128):
        pltpu.sync_copy(zrow_v, deg_sp.at[pl.ds(base + k * 128, 128)])
    pltpu.sync_copy(zrow_v.at[pl.ds(0, 64)],
                    deg_sp.at[pl.ds(base + 256, 64)])
    plsc.subcore_barrier()

    pltpu.sync_copy(idx_hbm.at[wid, pl.ds(0, 2)], ibuf.at[0])

    def body(jj, _):
        for b in range(2):
            j = jj * 2 + b

            @pl.when(j > 0)
            def _wait():
                pltpu.make_async_copy(
                    idx_hbm.at[wid, pl.ds(j * 2, 2)], ibuf.at[b],
                    sis[b]).wait()
                for k in range(2):
                    pltpu.make_async_copy(
                        ones_v, deg_sp.at[ibuf.at[1 - b, k, 1]],
                        sss[1 - b]).wait()

            @pl.when(j + 1 < CH // 2)
            def _issue():
                pltpu.async_copy(
                    idx_hbm.at[wid, pl.ds((j + 1) * 2, 2)], ibuf.at[1 - b],
                    sis[1 - b])

            for k in range(2):
                pltpu.async_copy(ones_v, deg_sp.at[ibuf.at[b, k, 1]],
                                 sss[b], add=True)
        return 0

    lax.fori_loop(0, CH // 4, body, 0)
    for k in range(2):
        pltpu.make_async_copy(
            ones_v, deg_sp.at[ibuf.at[1, k, 1]], sss[1]).wait()
    plsc.subcore_barrier()

    out_base = pl.multiple_of(sid * RPT, 8)
    pltpu.sync_copy(deg_sp.at[pl.ds(out_base, RPT)],
                    deg_hbm.at[cid, pl.ds(out_base, RPT)])


@functools.partial(
    pl.kernel,
    out_type=jax.ShapeDtypeStruct((NT2, D), jnp.float32),
    mesh=_mesh,
    scratch_types=[
        pltpu.VMEM((3, 2, 128), jnp.int32),
        pltpu.VMEM((2, C, D), jnp.float32),
        pltpu.VMEM((RPT // 8, 128), jnp.float32),
        pltpu.VMEM_SHARED((NHP, D), jnp.float32),
        pltpu.VMEM_SHARED((NHP, D), jnp.float32),
        pltpu.SemaphoreType.DMA,
        pltpu.SemaphoreType.DMA,
        pltpu.SemaphoreType.DMA,
        pltpu.SemaphoreType.DMA,
        pltpu.SemaphoreType.DMA,
        pltpu.SemaphoreType.DMA,
    ],
)
def _k1_propagate(z_hbm, idx_hbm, deg_hbm, out_hbm, ibuf, rows_v, degv,
                  zsp, acc_sp, si0, si1, sg0, sg1, ss0, ss1):
    cid = lax.axis_index("c")
    sid = lax.axis_index("s")
    wid = cid * NS + sid
    sis = (si0, si1)
    sgs = (sg0, sg1)
    sss = (ss0, ss1)

    base = pl.multiple_of(sid * RPT, 8)

    src_off = pl.multiple_of((1 - cid) * NHP + base, 8)
    stage_cp = pltpu.async_copy(
        z_hbm.at[pl.ds(src_off, RPT)], zsp.at[pl.ds(base, RPT)], sg0)
    dst_off = pl.multiple_of(cid * NHP + base, 8)
    deg_cp = pltpu.async_copy(
        deg_hbm.at[pl.ds(pl.multiple_of(dst_off // 8, 8), RPT // 8)], degv,
        sg1)

    zv = jnp.zeros((16,), jnp.float32)

    def zbody(i, _):
        r = i // (D // 16)
        col = (i % (D // 16)) * 16
        rows_v[0, r, pl.ds(col, 16)] = zv
        return 0
    lax.fori_loop(0, C * (D // 16), zbody, 0)

    for k in range(RPT // C):
        pltpu.sync_copy(rows_v.at[0], acc_sp.at[pl.ds(base + k * C, C)])
    pltpu.sync_copy(rows_v.at[0, pl.ds(0, RPT % C)],
                    acc_sp.at[pl.ds(base + (RPT // C) * C, RPT % C)])
    stage_cp.wait()
    deg_cp.wait()
    plsc.subcore_barrier()

    pltpu.sync_copy(idx_hbm.at[wid, 0], ibuf.at[0])
    pltpu.async_copy(zsp.at[ibuf.at[0, 0]], rows_v.at[0], sg0)
    pltpu.async_copy(idx_hbm.at[wid, 1], ibuf.at[1], si1)

    def body(jj, _):
        for b in range(2):
            j = jj * 2 + b

            @pl.when(j > 0)
            def _drain_scatter():
                pltpu.make_async_copy(
                    rows_v.at[1 - b], acc_sp.at[ibuf.at[(j - 1) % 3, 1]],
                    sss[1 - b]).wait()

            @pl.when(j + 1 < CH)
            def _gather_next():
                pltpu.make_async_copy(
                    idx_hbm.at[wid, j + 1], ibuf.at[(j + 1) % 3],
                    sis[1 - b]).wait()
                pltpu.async_copy(
                    zsp.at[ibuf.at[(j + 1) % 3, 0]], rows_v.at[1 - b],
                    sgs[1 - b])

                @pl.when(j + 2 < CH)
                def _prefetch_idx():
                    pltpu.async_copy(idx_hbm.at[wid, j + 2],
                                     ibuf.at[(j + 2) % 3], sis[b])

            pltpu.make_async_copy(
                zsp.at[ibuf.at[j % 3, 0]], rows_v.at[b], sgs[b]).wait()
            pltpu.async_copy(rows_v.at[b], acc_sp.at[ibuf.at[j % 3, 1]],
                             sss[b], add=True)
        return 0

    lax.fori_loop(0, CH // 2, body, 0)
    pltpu.make_async_copy(
        rows_v.at[1], acc_sp.at[ibuf.at[(CH - 1) % 3, 1]], sss[1]).wait()
    plsc.subcore_barrier()

    batches = ((0, C), (C, C), (2 * C, RPT - 2 * C))
    dsems = (sg0, sg1)
    osems = (ss0, ss1)

    def _drain_in(t):
        off, n = batches[t]
        return pltpu.async_copy(acc_sp.at[pl.ds(base + off, n)],
                                rows_v.at[t % 2, pl.ds(0, n)], dsems[t % 2])

    _drain_in(0)
    _drain_in(1)
    for t, (off, n) in enumerate(batches):
        buf = t % 2
        pltpu.make_async_copy(acc_sp.at[pl.ds(base + off, n)],
                              rows_v.at[buf, pl.ds(0, n)], dsems[buf]).wait()

        def sbody(r, _):
            r2 = off + r
            s = 1.0 / (degv[r2 // 8, pl.ds((r2 % 8) * 16, 16)] + EPS)
            for c in range(D // 16):
                rows_v[buf, r, pl.ds(c * 16, 16)] = (
                    rows_v[buf, r, pl.ds(c * 16, 16)] * s)
            return 0
        lax.fori_loop(0, n, sbody, 0)
        pltpu.async_copy(rows_v.at[buf, pl.ds(0, n)],
                         out_hbm.at[pl.ds(dst_off + off, n)], osems[buf])
        if t == 0:
            pass
        elif t == 1:
            o0, n0 = batches[0]
            pltpu.make_async_copy(
                rows_v.at[0, pl.ds(0, n0)],
                out_hbm.at[pl.ds(dst_off + o0, n0)], osems[0]).wait()
            _drain_in(2)
    for t in (1, 2):
        off, n = batches[t]
        pltpu.make_async_copy(rows_v.at[t % 2, pl.ds(0, n)],
                              out_hbm.at[pl.ds(dst_off + off, n)],
                              osems[t % 2]).wait()


_BR = 1280
_row_spec = pl.BlockSpec((_BR, D), lambda i: (i, 0))
_deg_spec = pl.BlockSpec((_BR, 16), lambda i: (i, 0))


def _deg_col(deg_ref):
    return deg_ref[:, 0:1] + EPS


def _tc_scale_body(x_ref, deg_ref, o_ref):
    o_ref[...] = x_ref[...] * lax.rsqrt(_deg_col(deg_ref))


_tc_scale = pl.pallas_call(
    _tc_scale_body,
    grid=(NT2 // _BR,),
    in_specs=[_row_spec, _deg_spec],
    out_specs=_row_spec,
    out_shape=jax.ShapeDtypeStruct((NT2, D), jnp.float32),
)


def _tc_final_body(z0_ref, z1_ref, z2_ref, z3_ref, deg_ref, o_ref):
    o_ref[...] = ((z0_ref[...] + z1_ref[...] + z2_ref[...] + z3_ref[...])
                  * (0.25 * lax.sqrt(_deg_col(deg_ref))))


_tc_final = pl.pallas_call(
    _tc_final_body,
    grid=(NT2 // _BR,),
    in_specs=[_row_spec, _row_spec, _row_spec, _row_spec, _deg_spec],
    out_specs=_row_spec,
    out_shape=jax.ShapeDtypeStruct((NT2, D), jnp.float32),
)


def kernel(user_emb, item_emb, edge_index):
    src = edge_index[0]
    dst = edge_index[1]

    srcH = jnp.concatenate([src[EH:] - NH, src[:EH]]).reshape(NW, EPW)
    dstH = jnp.concatenate([dst[EH:], dst[:EH] - NH]).reshape(NW, EPW)
    pad = CH * C - EPW
    src2 = jnp.pad(srcH, ((0, 0), (0, pad)), constant_values=0)
    dst2 = jnp.pad(dstH, ((0, 0), (0, pad)), constant_values=DROW)
    idx = jnp.stack([src2.reshape(NW, CH, 128),
                     dst2.reshape(NW, CH, 128)], axis=2)

    deg16 = _k0_degrees(idx).reshape(NT2, 16)

    padrows = jnp.zeros((NHP - NH, D), jnp.float32)
    all_emb = jnp.concatenate([user_emb, padrows, item_emb, padrows], axis=0)
    z0 = _tc_scale(all_emb, deg16)

    deg2d = deg16.reshape(NT2 // 8, 8 * 16)
    zs = [z0]
    for _ in range(N_LAYERS):
        zs.append(_k1_propagate(zs[-1], idx, deg2d))

    out = _tc_final(zs[0], zs[1], zs[2], zs[3], deg16)

    return (out[:N_USERS], out[NHP:NHP + N_ITEMS])
